# Initial kernel scaffold; baseline (speedup 1.0000x reference)
import functools

import jax
import jax.numpy as jnp
from jax.experimental import pallas as pl
from jax.experimental.pallas import tpu as pltpu

_N = 10000
_JB = 400
_IB = 1000
_RB = 1000


def _tf_body(x_ref, w_ref, al_ref, ar_ref, f_ref, el_ref, er_ref):
    f = jnp.dot(x_ref[...], w_ref[...], preferred_element_type=jnp.float32)
    f_ref[...] = f
    el_ref[...] = jnp.dot(f, al_ref[...], preferred_element_type=jnp.float32)
    er_ref[...] = jnp.dot(f, ar_ref[...], preferred_element_type=jnp.float32)


def _transform(h, W, AL, AR):
    n = h.shape[0]
    kin = h.shape[1]
    kout = W.shape[1]
    hh = AL.shape[1]
    grid = (n // _RB,)
    return pl.pallas_call(
        _tf_body,
        grid=grid,
        in_specs=[
            pl.BlockSpec((_RB, kin), lambda i: (i, 0)),
            pl.BlockSpec((kin, kout), lambda i: (0, 0)),
            pl.BlockSpec((kout, hh), lambda i: (0, 0)),
            pl.BlockSpec((kout, hh), lambda i: (0, 0)),
        ],
        out_specs=[
            pl.BlockSpec((_RB, kout), lambda i: (i, 0)),
            pl.BlockSpec((_RB, hh), lambda i: (i, 0)),
            pl.BlockSpec((_RB, hh), lambda i: (i, 0)),
        ],
        out_shape=[
            jax.ShapeDtypeStruct((n, kout), jnp.float32),
            jax.ShapeDtypeStruct((n, hh), jnp.float32),
            jax.ShapeDtypeStruct((n, hh), jnp.float32),
        ],
    )(h, W, AL, AR)


def _mp_body(adj_ref, elT_ref, er_ref, feat_ref, bias_ref, out_ref,
             num_ref, den_ref, *, H, D, nI, layer):
    i = pl.program_id(1)

    @pl.when(i == 0)
    def _init():
        num_ref[...] = jnp.zeros_like(num_ref)
        den_ref[...] = jnp.zeros_like(den_ref)

    adjb = adj_ref[...]
    elT = elT_ref[...]
    er = er_ref[...]
    A = jnp.exp(elT)
    C = jnp.exp(0.2 * elT)
    B = jnp.exp(er)
    Dd = jnp.exp(0.2 * er)
    for h in range(H):
        a = A[h:h + 1, :]
        c = C[h:h + 1, :]
        b = B[:, h:h + 1]
        d = Dd[:, h:h + 1]
        w = adjb * jnp.maximum(b * a, d * c)
        num_ref[:, h * D:(h + 1) * D] += jnp.dot(
            w, feat_ref[:, h * D:(h + 1) * D], preferred_element_type=jnp.float32)
        den_ref[:, h:h + 1] += jnp.sum(w, axis=1, keepdims=True)

    @pl.when(i == nI - 1)
    def _final():
        num = num_ref[...]
        den = den_ref[...]
        pieces = []
        for h in range(H):
            dh = den[:, h:h + 1]
            safe = jnp.where(dh > 0, dh, 1.0)
            pieces.append(num[:, h * D:(h + 1) * D] / safe)
        res = jnp.concatenate(pieces, axis=1) + bias_ref[...]
        if layer == 1:
            out_ref[...] = jnp.where(res > 0, res,
                                     jnp.exp(jnp.minimum(res, 0.0)) - 1.0)
        else:
            lg = res[:, 0:2]
            m = jnp.max(lg, axis=1, keepdims=True)
            lse = m + jnp.log(jnp.sum(jnp.exp(lg - m), axis=1, keepdims=True))
            out_ref[...] = lg - lse


def _message_pass(adj, elT, er, feat, bias, H, D, layer):
    n = adj.shape[0]
    nJ = n // _JB
    nI = n // _IB
    hd = feat.shape[1]
    out_c = 128 if layer == 1 else 2
    body = functools.partial(_mp_body, H=H, D=D, nI=nI, layer=layer)
    return pl.pallas_call(
        body,
        grid=(nJ, nI),
        in_specs=[
            pl.BlockSpec((_JB, _IB), lambda j, i: (j, i)),
            pl.BlockSpec((8, _IB), lambda j, i: (0, i)),
            pl.BlockSpec((_JB, 8), lambda j, i: (j, 0)),
            pl.BlockSpec((_IB, hd), lambda j, i: (i, 0)),
            pl.BlockSpec((1, hd), lambda j, i: (0, 0)),
        ],
        out_specs=pl.BlockSpec((_JB, out_c), lambda j, i: (j, 0)),
        out_shape=jax.ShapeDtypeStruct((n, out_c), jnp.float32),
        scratch_shapes=[
            pltpu.VMEM((_JB, hd), jnp.float32),
            pltpu.VMEM((_JB, 8), jnp.float32),
        ],
        compiler_params=pltpu.CompilerParams(
            dimension_semantics=("parallel", "arbitrary"),
        ),
    )(adj, elT, er, feat, bias)


def kernel(x, adj, W1, al1, ar1, b1, W2, al2, ar2, b2):
    eye = jnp.eye(8, dtype=jnp.float32)
    AL1 = (al1.reshape(8, 16)[:, :, None] * eye[:, None, :]).reshape(128, 8)
    AR1 = (ar1.reshape(8, 16)[:, :, None] * eye[:, None, :]).reshape(128, 8)
    W2p = jnp.pad(W2, ((0, 0), (0, 126)))
    AL2 = jnp.pad(al2.reshape(2, 1), ((0, 126), (0, 7)))
    AR2 = jnp.pad(ar2.reshape(2, 1), ((0, 126), (0, 7)))
    b1r = b1.reshape(1, 128)
    b2p = jnp.pad(b2, (0, 126)).reshape(1, 128)

    f1, el1, er1 = _transform(x, W1, AL1, AR1)
    h1 = _message_pass(adj, el1.T, er1, f1, b1r, H=8, D=16, layer=1)
    f2, el2, er2 = _transform(h1, W2p, AL2, AR2)
    out = _message_pass(adj, el2.T, er2, f2, b2p, H=1, D=128, layer=2)
    return out


# speedup vs baseline: 37.9545x; 37.9545x over previous
import functools

import jax
import jax.numpy as jnp
from jax.experimental import pallas as pl
from jax.experimental.pallas import tpu as pltpu

_N = 10000
_JB = 200
_RB = 1000


def _tf_body(x_ref, w_ref, al_ref, ar_ref, f_ref, el_ref, er_ref):
    f = jnp.dot(x_ref[...], w_ref[...], preferred_element_type=jnp.float32)
    f_ref[...] = f
    el_ref[...] = jnp.dot(f, al_ref[...], preferred_element_type=jnp.float32)
    er_ref[...] = jnp.dot(f, ar_ref[...], preferred_element_type=jnp.float32)


def _transform(h, W, AL, AR):
    n = h.shape[0]
    kin = h.shape[1]
    kout = W.shape[1]
    hh = AL.shape[1]
    grid = (n // _RB,)
    return pl.pallas_call(
        _tf_body,
        grid=grid,
        in_specs=[
            pl.BlockSpec((_RB, kin), lambda i: (i, 0)),
            pl.BlockSpec((kin, kout), lambda i: (0, 0)),
            pl.BlockSpec((kout, hh), lambda i: (0, 0)),
            pl.BlockSpec((kout, hh), lambda i: (0, 0)),
        ],
        out_specs=[
            pl.BlockSpec((_RB, kout), lambda i: (i, 0)),
            pl.BlockSpec((_RB, hh), lambda i: (i, 0)),
            pl.BlockSpec((_RB, hh), lambda i: (i, 0)),
        ],
        out_shape=[
            jax.ShapeDtypeStruct((n, kout), jnp.float32),
            jax.ShapeDtypeStruct((n, hh), jnp.float32),
            jax.ShapeDtypeStruct((n, hh), jnp.float32),
        ],
    )(h, W, AL, AR)


def _mp_body(adj_ref, elT_ref, er_ref, feat_ref, bias_ref, out_ref, *, H, D, layer):
    adjb = adj_ref[...]
    elT = elT_ref[...]
    er = er_ref[...]
    A = jnp.exp(elT)
    C = jnp.exp(0.2 * elT)
    B = jnp.exp(er)
    Dd = jnp.exp(0.2 * er)
    nums = []
    for h in range(H):
        a = A[h:h + 1, :]
        c = C[h:h + 1, :]
        b = B[:, h:h + 1]
        d = Dd[:, h:h + 1]
        w = adjb * jnp.maximum(b * a, d * c)
        num = jnp.dot(w, feat_ref[:, h * D:(h + 1) * D],
                      preferred_element_type=jnp.float32)
        den = jnp.sum(w, axis=1, keepdims=True)
        safe = jnp.where(den > 0, den, 1.0)
        nums.append(num / safe)
    res = jnp.concatenate(nums, axis=1) + bias_ref[...]
    if layer == 1:
        out_ref[...] = jnp.where(res > 0, res,
                                 jnp.exp(jnp.minimum(res, 0.0)) - 1.0)
    else:
        lg = res[:, 0:2]
        m = jnp.max(lg, axis=1, keepdims=True)
        lse = m + jnp.log(jnp.sum(jnp.exp(lg - m), axis=1, keepdims=True))
        out_ref[...] = lg - lse


def _message_pass(adj, elT, er, feat, bias, H, D, layer):
    n = adj.shape[0]
    nJ = n // _JB
    hd = feat.shape[1]
    out_c = 128 if layer == 1 else 2
    body = functools.partial(_mp_body, H=H, D=D, layer=layer)
    return pl.pallas_call(
        body,
        grid=(nJ,),
        in_specs=[
            pl.BlockSpec((_JB, n), lambda j: (j, 0)),
            pl.BlockSpec((8, n), lambda j: (0, 0)),
            pl.BlockSpec((_JB, 8), lambda j: (j, 0)),
            pl.BlockSpec((n, hd), lambda j: (0, 0)),
            pl.BlockSpec((1, hd), lambda j: (0, 0)),
        ],
        out_specs=pl.BlockSpec((_JB, out_c), lambda j: (j, 0)),
        out_shape=jax.ShapeDtypeStruct((n, out_c), jnp.float32),
        compiler_params=pltpu.CompilerParams(
            dimension_semantics=("arbitrary",),
        ),
    )(adj, elT, er, feat, bias)


def kernel(x, adj, W1, al1, ar1, b1, W2, al2, ar2, b2):
    eye = jnp.eye(8, dtype=jnp.float32)
    AL1 = (al1.reshape(8, 16)[:, :, None] * eye[:, None, :]).reshape(128, 8)
    AR1 = (ar1.reshape(8, 16)[:, :, None] * eye[:, None, :]).reshape(128, 8)
    W2p = jnp.pad(W2, ((0, 0), (0, 126)))
    AL2 = jnp.pad(al2.reshape(2, 1), ((0, 126), (0, 7)))
    AR2 = jnp.pad(ar2.reshape(2, 1), ((0, 126), (0, 7)))
    b1r = b1.reshape(1, 128)
    b2p = jnp.pad(b2, (0, 126)).reshape(1, 128)

    f1, el1, er1 = _transform(x, W1, AL1, AR1)
    h1 = _message_pass(adj, el1.T, er1, f1, b1r, H=8, D=16, layer=1)
    f2, el2, er2 = _transform(h1, W2p, AL2, AR2)
    out = _message_pass(adj, el2.T, er2, f2, b2p, H=1, D=128, layer=2)
    return out


# speedup vs baseline: 40.8202x; 1.0755x over previous
import functools

import jax
import jax.numpy as jnp
from jax.experimental import pallas as pl
from jax.experimental.pallas import tpu as pltpu

_N = 10000
_JB = 200
_RB = 1000


def _tf_body(x_ref, w_ref, al_ref, ar_ref, f_ref, el_ref, er_ref):
    f = jnp.dot(x_ref[...], w_ref[...], preferred_element_type=jnp.float32)
    f_ref[...] = f
    el_ref[...] = jnp.dot(f, al_ref[...], preferred_element_type=jnp.float32)
    er_ref[...] = jnp.dot(f, ar_ref[...], preferred_element_type=jnp.float32)


def _transform(h, W, AL, AR):
    n = h.shape[0]
    kin = h.shape[1]
    kout = W.shape[1]
    hh = AL.shape[1]
    grid = (n // _RB,)
    return pl.pallas_call(
        _tf_body,
        grid=grid,
        in_specs=[
            pl.BlockSpec((_RB, kin), lambda i: (i, 0)),
            pl.BlockSpec((kin, kout), lambda i: (0, 0)),
            pl.BlockSpec((kout, hh), lambda i: (0, 0)),
            pl.BlockSpec((kout, hh), lambda i: (0, 0)),
        ],
        out_specs=[
            pl.BlockSpec((_RB, kout), lambda i: (i, 0)),
            pl.BlockSpec((_RB, hh), lambda i: (i, 0)),
            pl.BlockSpec((_RB, hh), lambda i: (i, 0)),
        ],
        out_shape=[
            jax.ShapeDtypeStruct((n, kout), jnp.float32),
            jax.ShapeDtypeStruct((n, hh), jnp.float32),
            jax.ShapeDtypeStruct((n, hh), jnp.float32),
        ],
    )(h, W, AL, AR)


def _mp_body(adj_ref, elT_ref, er_ref, feat_ref, bias_ref, out_ref, *, H, D, layer):
    adjb = adj_ref[...]
    elT = elT_ref[...]
    er = er_ref[...]
    A = jnp.exp(elT)
    C = jnp.exp(0.2 * elT)
    B = jnp.exp(er)
    Dd = jnp.exp(0.2 * er)
    fbf = feat_ref[...].astype(jnp.bfloat16)
    nums = []
    for h in range(H):
        a = A[h:h + 1, :]
        c = C[h:h + 1, :]
        b = B[:, h:h + 1]
        d = Dd[:, h:h + 1]
        w = adjb * jnp.maximum(b * a, d * c)
        num = jnp.dot(w.astype(jnp.bfloat16), fbf[:, h * D:(h + 1) * D],
                      preferred_element_type=jnp.float32)
        den = jnp.sum(w, axis=1, keepdims=True)
        safe = jnp.where(den > 0, den, 1.0)
        nums.append(num / safe)
    res = jnp.concatenate(nums, axis=1) + bias_ref[...]
    if layer == 1:
        out_ref[...] = jnp.where(res > 0, res,
                                 jnp.exp(jnp.minimum(res, 0.0)) - 1.0)
    else:
        lg = res[:, 0:2]
        m = jnp.max(lg, axis=1, keepdims=True)
        lse = m + jnp.log(jnp.sum(jnp.exp(lg - m), axis=1, keepdims=True))
        out_ref[...] = lg - lse


def _message_pass(adj, elT, er, feat, bias, H, D, layer):
    n = adj.shape[0]
    nJ = n // _JB
    hd = feat.shape[1]
    out_c = 128 if layer == 1 else 2
    body = functools.partial(_mp_body, H=H, D=D, layer=layer)
    return pl.pallas_call(
        body,
        grid=(nJ,),
        in_specs=[
            pl.BlockSpec((_JB, n), lambda j: (j, 0)),
            pl.BlockSpec((8, n), lambda j: (0, 0)),
            pl.BlockSpec((_JB, 8), lambda j: (j, 0)),
            pl.BlockSpec((n, hd), lambda j: (0, 0)),
            pl.BlockSpec((1, hd), lambda j: (0, 0)),
        ],
        out_specs=pl.BlockSpec((_JB, out_c), lambda j: (j, 0)),
        out_shape=jax.ShapeDtypeStruct((n, out_c), jnp.float32),
        compiler_params=pltpu.CompilerParams(
            dimension_semantics=("arbitrary",),
        ),
    )(adj, elT, er, feat, bias)


def kernel(x, adj, W1, al1, ar1, b1, W2, al2, ar2, b2):
    eye = jnp.eye(8, dtype=jnp.float32)
    AL1 = (al1.reshape(8, 16)[:, :, None] * eye[:, None, :]).reshape(128, 8)
    AR1 = (ar1.reshape(8, 16)[:, :, None] * eye[:, None, :]).reshape(128, 8)
    W2p = jnp.pad(W2, ((0, 0), (0, 126)))
    AL2 = jnp.pad(al2.reshape(2, 1), ((0, 126), (0, 7)))
    AR2 = jnp.pad(ar2.reshape(2, 1), ((0, 126), (0, 7)))
    b1r = b1.reshape(1, 128)
    b2p = jnp.pad(b2, (0, 126)).reshape(1, 128)

    f1, el1, er1 = _transform(x, W1, AL1, AR1)
    h1 = _message_pass(adj, el1.T, er1, f1, b1r, H=8, D=16, layer=1)
    f2, el2, er2 = _transform(h1, W2p, AL2, AR2)
    out = _message_pass(adj, el2.T, er2, f2, b2p, H=1, D=128, layer=2)
    return out


# speedup vs baseline: 70.8733x; 1.7362x over previous
import functools

import jax
import jax.numpy as jnp
from jax.experimental import pallas as pl
from jax.experimental.pallas import tpu as pltpu

_JB = 200
_RB = 1000


def _tf_body(x_ref, w_ref, al_ref, ar_ref, f_ref, el_ref, er_ref, *, H, D):
    f = jnp.dot(x_ref[...], w_ref[...], preferred_element_type=jnp.float32)
    el_ref[...] = jnp.dot(f, al_ref[...], preferred_element_type=jnp.float32)
    er_ref[...] = jnp.dot(f, ar_ref[...], preferred_element_type=jnp.float32)
    ones = jnp.ones((f.shape[0], 1), dtype=jnp.bfloat16)
    fb = f.astype(jnp.bfloat16)
    pieces = []
    for h in range(H):
        pieces.append(fb[:, h * D:(h + 1) * D])
        pieces.append(ones)
    f_ref[...] = jnp.concatenate(pieces, axis=1)


def _transform(h, W, AL, AR, H, D):
    n = h.shape[0]
    kin = h.shape[1]
    kout = W.shape[1]
    hh = AL.shape[1]
    body = functools.partial(_tf_body, H=H, D=D)
    return pl.pallas_call(
        body,
        grid=(n // _RB,),
        in_specs=[
            pl.BlockSpec((_RB, kin), lambda i: (i, 0)),
            pl.BlockSpec((kin, kout), lambda i: (0, 0)),
            pl.BlockSpec((kout, hh), lambda i: (0, 0)),
            pl.BlockSpec((kout, hh), lambda i: (0, 0)),
        ],
        out_specs=[
            pl.BlockSpec((_RB, H * (D + 1)), lambda i: (i, 0)),
            pl.BlockSpec((_RB, hh), lambda i: (i, 0)),
            pl.BlockSpec((_RB, hh), lambda i: (i, 0)),
        ],
        out_shape=[
            jax.ShapeDtypeStruct((n, H * (D + 1)), jnp.bfloat16),
            jax.ShapeDtypeStruct((n, hh), jnp.float32),
            jax.ShapeDtypeStruct((n, hh), jnp.float32),
        ],
    )(h, W, AL, AR)


def _mp_body(adj_ref, elT_ref, er_ref, feat_ref, bias_ref, out_ref, *, H, D, layer):
    adjb = adj_ref[...].astype(jnp.bfloat16)
    elT = elT_ref[...]
    er = er_ref[...]
    A = jnp.exp(elT).astype(jnp.bfloat16)
    C = jnp.exp(0.2 * elT).astype(jnp.bfloat16)
    B = jnp.exp(er).astype(jnp.bfloat16)
    Dd = jnp.exp(0.2 * er).astype(jnp.bfloat16)
    fbf = feat_ref[...]
    nums = []
    for h in range(H):
        a = A[h:h + 1, :]
        c = C[h:h + 1, :]
        b = B[:, h:h + 1]
        d = Dd[:, h:h + 1]
        w = adjb * jnp.maximum(b * a, d * c)
        nd = jnp.dot(w, fbf[:, h * (D + 1):(h + 1) * (D + 1)],
                     preferred_element_type=jnp.float32)
        den = nd[:, D:D + 1]
        safe = jnp.where(den > 0, den, 1.0)
        nums.append(nd[:, 0:D] / safe)
    res = jnp.concatenate(nums, axis=1) + bias_ref[...]
    if layer == 1:
        out_ref[...] = jnp.where(res > 0, res,
                                 jnp.exp(jnp.minimum(res, 0.0)) - 1.0)
    else:
        m = jnp.max(res, axis=1, keepdims=True)
        lse = m + jnp.log(jnp.sum(jnp.exp(res - m), axis=1, keepdims=True))
        out_ref[...] = res - lse


def _message_pass(adj, elT, er, feat, bias, H, D, layer):
    n = adj.shape[0]
    hd = feat.shape[1]
    out_c = H * D
    body = functools.partial(_mp_body, H=H, D=D, layer=layer)
    return pl.pallas_call(
        body,
        grid=(n // _JB,),
        in_specs=[
            pl.BlockSpec((_JB, n), lambda j: (j, 0)),
            pl.BlockSpec((8, n), lambda j: (0, 0)),
            pl.BlockSpec((_JB, 8), lambda j: (j, 0)),
            pl.BlockSpec((n, hd), lambda j: (0, 0)),
            pl.BlockSpec((1, out_c), lambda j: (0, 0)),
        ],
        out_specs=pl.BlockSpec((_JB, out_c), lambda j: (j, 0)),
        out_shape=jax.ShapeDtypeStruct((n, out_c), jnp.float32),
        compiler_params=pltpu.CompilerParams(
            dimension_semantics=("arbitrary",),
        ),
    )(adj, elT, er, feat, bias)


def kernel(x, adj, W1, al1, ar1, b1, W2, al2, ar2, b2):
    eye = jnp.eye(8, dtype=jnp.float32)
    AL1 = (al1.reshape(8, 16)[:, :, None] * eye[:, None, :]).reshape(128, 8)
    AR1 = (ar1.reshape(8, 16)[:, :, None] * eye[:, None, :]).reshape(128, 8)
    AL2 = jnp.pad(al2.reshape(2, 1), ((0, 0), (0, 7)))
    AR2 = jnp.pad(ar2.reshape(2, 1), ((0, 0), (0, 7)))
    b1r = b1.reshape(1, 128)
    b2r = b2.reshape(1, 2)

    f1, el1, er1 = _transform(x, W1, AL1, AR1, H=8, D=16)
    h1 = _message_pass(adj, el1.T, er1, f1, b1r, H=8, D=16, layer=1)
    f2, el2, er2 = _transform(h1, W2, AL2, AR2, H=1, D=2)
    out = _message_pass(adj, el2.T, er2, f2, b2r, H=1, D=2, layer=2)
    return out
